# baseline (speedup 1.0000x reference)
import jax
import jax.numpy as jnp
from jax import lax
from jax.experimental import pallas as pl
from jax.experimental.pallas import tpu as pltpu
from jax.experimental.pallas import tpu_sc as plsc

E = 8
K = 2
N = 2048
D = 768
F = 768
EP = 128
R = N * K
T = 128
P = R + E * T
NT = P // T
NTP = 48

NC = 2
NS = 16
L = 16


def _gating_kernel(data_ref, wg_ref, a1_ref, a2_ref, g1_ref, g2_ref,
                   loss_ref):
    x = data_ref[...]
    logits = jnp.dot(x, wg_ref[...], preferred_element_type=jnp.float32)
    lane = lax.broadcasted_iota(jnp.int32, (N, EP), 1)
    neg = jnp.float32(-jnp.inf)
    logits = jnp.where(lane < E, logits, neg)
    l1 = jnp.max(logits, axis=1, keepdims=True)
    a1 = jnp.min(jnp.where(logits == l1, lane, EP), axis=1, keepdims=True)
    m = jnp.where(lane == a1, neg, logits)
    l2 = jnp.max(m, axis=1, keepdims=True)
    a2 = jnp.min(jnp.where(m == l2, lane, EP), axis=1, keepdims=True)
    e2 = jnp.exp(l2 - l1)
    denom = 1.0 + e2
    g1 = 1.0 / denom
    g2 = e2 / denom
    a1_ref[...] = a1
    a2_ref[...] = a2
    g1_ref[...] = g1
    g2_ref[...] = g2
    gates = (jnp.where(lane == a1, g1, 0.0)
             + jnp.where(lane == a2, g2, 0.0))
    lane_m = (lane[0:1, :] < E).astype(jnp.float32)
    importance = jnp.sum(gates, axis=0, keepdims=True) * lane_m
    load = jnp.sum((gates > 0.0).astype(jnp.float32), axis=0,
                   keepdims=True) * lane_m

    def cv2(v):
        mean = jnp.sum(v) / E
        var = jnp.sum(jnp.where(lane_m > 0, (v - mean) ** 2, 0.0)) / (E - 1)
        return var / (mean * mean + 1e-10)

    loss_ref[0, 0] = (cv2(importance) + cv2(load)) * 0.01


def _gating(data, w_gate_p):
    return pl.pallas_call(
        _gating_kernel,
        out_specs=[
            pl.BlockSpec((N, 1), lambda: (0, 0)),
            pl.BlockSpec((N, 1), lambda: (0, 0)),
            pl.BlockSpec((N, 1), lambda: (0, 0)),
            pl.BlockSpec((N, 1), lambda: (0, 0)),
            pl.BlockSpec(memory_space=pltpu.SMEM),
        ],
        out_shape=[
            jax.ShapeDtypeStruct((N, 1), jnp.int32),
            jax.ShapeDtypeStruct((N, 1), jnp.int32),
            jax.ShapeDtypeStruct((N, 1), jnp.float32),
            jax.ShapeDtypeStruct((N, 1), jnp.float32),
            jax.ShapeDtypeStruct((1, 1), jnp.float32),
        ],
    )(data, w_gate_p)


TPW = N // NS
NCH = TPW // L


def _routing_body(a1_hbm, a2_hbm, g1_hbm, g2_hbm,
                  st_hbm, sg_hbm, pf_hbm, te_hbm,
                  e1_v, e2_v, g1_v, g2_v, tok_v, pos0_v, pos1_v,
                  cnt_v, all_cnt_v, te_v, shared_cnt, sem):
    wid = lax.axis_index("s")
    t0 = wid * TPW
    pltpu.sync_copy(a1_hbm.at[pl.ds(t0, TPW)], e1_v)
    pltpu.sync_copy(a2_hbm.at[pl.ds(t0, TPW)], e2_v)
    pltpu.sync_copy(g1_hbm.at[pl.ds(t0, TPW)], g1_v)
    pltpu.sync_copy(g2_hbm.at[pl.ds(t0, TPW)], g2_v)

    lanes = jnp.arange(16)

    cnt = [jnp.int32(0)] * E
    for c in range(NCH):
        ids0 = e1_v[pl.ds(c * L, L)]
        ids1 = e2_v[pl.ds(c * L, L)]
        for e in range(E):
            p0 = plsc.cumsum(jnp.where(ids0 == e, 1, 0))
            p1 = plsc.cumsum(jnp.where(ids1 == e, 1, 0))
            cnt[e] = cnt[e] + p0[15] + p1[15]
    cv = jnp.zeros((16,), jnp.int32)
    for e in range(E):
        cv = jnp.where(lanes == e, cnt[e], cv)
    cnt_v[...] = cv

    pltpu.sync_copy(cnt_v, shared_cnt.at[pl.ds(wid * 16, 16)])
    plsc.subcore_barrier()
    pltpu.sync_copy(shared_cnt, all_cnt_v)

    totals = jnp.zeros((16,), jnp.int32)
    for w in range(NS):
        totals = totals + all_cnt_v[pl.ds(w * 16, 16)]

    def pre_body(w, acc):
        return acc + all_cnt_v[pl.ds(w * 16, 16)]

    prev = lax.fori_loop(0, wid, pre_body, jnp.zeros((16,), jnp.int32))

    pstart = [jnp.int32(0)] * (E + 1)
    for e in range(E):
        pe = ((totals[e] + T - 1) // T) * T
        pstart[e + 1] = pstart[e] + pe
    off = [pstart[e] + prev[e] for e in range(E)]

    for idv, posv in ((e1_v, pos0_v), (e2_v, pos1_v)):
        for c in range(NCH):
            ids = idv[pl.ds(c * L, L)]
            pos = jnp.zeros((16,), jnp.int32)
            for e in range(E):
                msk = ids == e
                incl = plsc.cumsum(jnp.where(msk, 1, 0))
                pos = jnp.where(msk, off[e] + incl - 1, pos)
                off[e] = off[e] + incl[15]
            posv[pl.ds(c * L, L)] = pos

    for c in range(NCH):
        tok_v[pl.ds(c * L, L)] = t0 + c * L + lanes

    pltpu.sync_copy(pos0_v, pf_hbm.at[pl.ds(t0, TPW)])
    pltpu.sync_copy(pos1_v, pf_hbm.at[pl.ds(N + t0, TPW)])
    pltpu.sync_copy(tok_v, st_hbm.at[pos0_v])
    pltpu.sync_copy(tok_v, st_hbm.at[pos1_v])
    pltpu.sync_copy(g1_v, sg_hbm.at[pos0_v])
    pltpu.sync_copy(g2_v, sg_hbm.at[pos1_v])

    @pl.when(wid == 0)
    def _tile_map():
        for c in range(NTP // L):
            tile_start = (c * L + lanes) * T
            g = jnp.zeros((16,), jnp.int32)
            for e in range(E):
                g = g + jnp.where(tile_start >= pstart[e + 1], 1, 0)
            te_v[pl.ds(c * L, L)] = jnp.minimum(g, E - 1)
        pltpu.sync_copy(te_v, te_hbm)


def _routing(a1, a2, g1, g2):
    mesh = plsc.VectorSubcoreMesh(core_axis_name="c", subcore_axis_name="s",
                                  num_cores=1)
    return pl.kernel(
        _routing_body,
        out_type=[
            jax.ShapeDtypeStruct((P,), jnp.int32),
            jax.ShapeDtypeStruct((P,), jnp.float32),
            jax.ShapeDtypeStruct((K * N,), jnp.int32),
            jax.ShapeDtypeStruct((NTP,), jnp.int32),
        ],
        mesh=mesh,
        scratch_types=[
            pltpu.VMEM((TPW,), jnp.int32),
            pltpu.VMEM((TPW,), jnp.int32),
            pltpu.VMEM((TPW,), jnp.float32),
            pltpu.VMEM((TPW,), jnp.float32),
            pltpu.VMEM((TPW,), jnp.int32),
            pltpu.VMEM((TPW,), jnp.int32),
            pltpu.VMEM((TPW,), jnp.int32),
            pltpu.VMEM((16,), jnp.int32),
            pltpu.VMEM((NS * 16,), jnp.int32),
            pltpu.VMEM((NTP,), jnp.int32),
            pltpu.VMEM_SHARED((NS * 16,), jnp.int32),
            pltpu.SemaphoreType.DMA,
        ],
        compiler_params=pltpu.CompilerParams(needs_layout_passes=False),
    )(a1, a2, g1, g2)


GPW = P // (NC * NS)
GCH = GPW // 2


def _gather_body(st_hbm, data_hbm, xs_hbm, idx_v, rows_v, sem):
    c = lax.axis_index("c")
    s = lax.axis_index("s")
    wid = s * NC + c
    for ch in range(2):
        base = wid * GPW + ch * GCH
        pltpu.sync_copy(st_hbm.at[pl.ds(base, GCH)], idx_v)
        for v in range(GCH // L):
            x = idx_v[pl.ds(v * L, L)]
            idx_v[pl.ds(v * L, L)] = jnp.minimum(jnp.maximum(x, 0), N - 1)
        pltpu.async_copy(data_hbm.at[idx_v], rows_v, sem).wait()
        pltpu.sync_copy(rows_v, xs_hbm.at[pl.ds(base, GCH)])


def _gather(sorted_tok, data):
    mesh = plsc.VectorSubcoreMesh(core_axis_name="c", subcore_axis_name="s")
    return pl.kernel(
        _gather_body,
        out_type=jax.ShapeDtypeStruct((P, D), jnp.float32),
        mesh=mesh,
        scratch_types=[
            pltpu.VMEM((GCH,), jnp.int32),
            pltpu.VMEM((GCH, D), jnp.float32),
            pltpu.SemaphoreType.DMA,
        ],
        compiler_params=pltpu.CompilerParams(needs_layout_passes=False),
    )(sorted_tok, data)


def _grouped_kernel(te_ref, xs_ref, w1_ref, b1_ref, w2_ref, b2_ref, g_ref,
                    out_ref):
    h = jnp.dot(xs_ref[...], w1_ref[0], preferred_element_type=jnp.float32)
    h = jnp.maximum(h + b1_ref[0], 0.0)
    o = jnp.dot(h, w2_ref[0], preferred_element_type=jnp.float32)
    out_ref[...] = (o + b2_ref[0]) * g_ref[...]


def _grouped(te, xs, W1, b1r, W2, b2r, sg2):
    grid_spec = pltpu.PrefetchScalarGridSpec(
        num_scalar_prefetch=1,
        grid=(NT,),
        in_specs=[
            pl.BlockSpec((T, D), lambda i, te: (i, 0)),
            pl.BlockSpec((1, D, F), lambda i, te: (te[i], 0, 0)),
            pl.BlockSpec((1, 1, F), lambda i, te: (te[i], 0, 0)),
            pl.BlockSpec((1, F, D), lambda i, te: (te[i], 0, 0)),
            pl.BlockSpec((1, 1, D), lambda i, te: (te[i], 0, 0)),
            pl.BlockSpec((T, 1), lambda i, te: (i, 0)),
        ],
        out_specs=pl.BlockSpec((T, D), lambda i, te: (i, 0)),
    )
    return pl.pallas_call(
        _grouped_kernel,
        grid_spec=grid_spec,
        out_shape=jax.ShapeDtypeStruct((P, D), jnp.float32),
        compiler_params=pltpu.CompilerParams(
            dimension_semantics=("arbitrary",),
        ),
    )(te, xs, W1, b1r, W2, b2r, sg2)


CTOK = N // (NC * NS)
CCH = CTOK // 2


def _combine_body(pf_hbm, so_hbm, y_hbm, pos_v, rows_v, y_v, sem):
    c = lax.axis_index("c")
    s = lax.axis_index("s")
    wid = s * NC + c
    for ch in range(2):
        tok0 = wid * CTOK + ch * CCH
        pltpu.sync_copy(pf_hbm.at[pl.ds(tok0, CCH)], pos_v)
        pltpu.async_copy(so_hbm.at[pos_v], y_v, sem).wait()
        pltpu.sync_copy(pf_hbm.at[pl.ds(N + tok0, CCH)], pos_v)
        pltpu.async_copy(so_hbm.at[pos_v], rows_v, sem).wait()

        def add_body(j, _):
            def lane_body(l, _2):
                a = y_v[j, pl.ds(l * L, L)]
                b = rows_v[j, pl.ds(l * L, L)]
                y_v[j, pl.ds(l * L, L)] = a + b
                return 0

            lax.fori_loop(0, D // L, lane_body, 0)
            return 0

        lax.fori_loop(0, CCH, add_body, 0)
        pltpu.sync_copy(y_v, y_hbm.at[pl.ds(tok0, CCH)])


def _combine(pos_flat, sorted_out):
    mesh = plsc.VectorSubcoreMesh(core_axis_name="c", subcore_axis_name="s")
    return pl.kernel(
        _combine_body,
        out_type=jax.ShapeDtypeStruct((N, D), jnp.float32),
        mesh=mesh,
        scratch_types=[
            pltpu.VMEM((CCH,), jnp.int32),
            pltpu.VMEM((CCH, D), jnp.float32),
            pltpu.VMEM((CCH, D), jnp.float32),
            pltpu.SemaphoreType.DMA,
        ],
        compiler_params=pltpu.CompilerParams(needs_layout_passes=False),
    )(pos_flat, sorted_out)


@jax.jit
def _moe(data, w_gate_p, W1, b1r, W2, b2r):
    a1, a2, g1, g2, loss = _gating(data, w_gate_p)
    st, sg, pf, te = _routing(a1.reshape(N), a2.reshape(N),
                              g1.reshape(N), g2.reshape(N))
    xs = _gather(st, data)
    so = _grouped(te, xs, W1, b1r, W2, b2r, sg.reshape(P, 1))
    y = _combine(pf, so)
    return y, loss[0, 0]


def kernel(data, w_gate, w_noise, W1, b1, W2, b2):
    del w_noise
    w_gate_p = jnp.pad(w_gate, ((0, 0), (0, EP - E)))
    return _moe(data, w_gate_p, W1, b1[:, None, :], W2, b2[:, None, :])


# speedup vs baseline: 2.8617x; 2.8617x over previous
import jax
import jax.numpy as jnp
from jax import lax
from jax.experimental import pallas as pl
from jax.experimental.pallas import tpu as pltpu

E = 8
K = 2
N = 2048
D = 768
F = 768
EP = 128
TN = 256
NTL = N // TN


def _moe_kernel(x32_ref, x16_ref, wg_ref, w1_ref, b1_ref, w2a_ref, b2p_ref,
                y_ref, loss_ref, h_ref, imp_ref, load_ref):
    t = pl.program_id(0)
    x32 = x32_ref[...]
    logits = jnp.dot(x32, wg_ref[...], preferred_element_type=jnp.float32)
    lane = lax.broadcasted_iota(jnp.int32, (TN, EP), 1)
    neg = jnp.float32(-jnp.inf)
    logits = jnp.where(lane < E, logits, neg)
    l1 = jnp.max(logits, axis=1, keepdims=True)
    a1 = jnp.min(jnp.where(logits == l1, lane, EP), axis=1, keepdims=True)
    m = jnp.where(lane == a1, neg, logits)
    l2 = jnp.max(m, axis=1, keepdims=True)
    a2 = jnp.min(jnp.where(m == l2, lane, EP), axis=1, keepdims=True)
    e2 = jnp.exp(l2 - l1)
    denom = 1.0 + e2
    g1 = 1.0 / denom
    g2 = e2 / denom
    gates = (jnp.where(lane == a1, g1, 0.0)
             + jnp.where(lane == a2, g2, 0.0))

    x16 = x16_ref[...]
    for e in range(E):
        h = jnp.dot(x16, w1_ref[e], preferred_element_type=jnp.float32)
        h = jnp.maximum(h + b1_ref[0, :, e * F:(e + 1) * F], 0.0)
        gcol = jnp.sum(jnp.where(lane == e, gates, 0.0), axis=1,
                       keepdims=True)
        h_ref[:, e * F:(e + 1) * F] = (h * gcol).astype(jnp.bfloat16)

    o = jnp.dot(h_ref[...], w2a_ref[...], preferred_element_type=jnp.float32)
    o = o + jnp.dot(gates, b2p_ref[...], preferred_element_type=jnp.float32)
    y_ref[...] = o

    lane_m = (lane[0:1, :] < E).astype(jnp.float32)
    imp_t = jnp.sum(gates, axis=0, keepdims=True) * lane_m
    load_t = jnp.sum((gates > 0.0).astype(jnp.float32), axis=0,
                     keepdims=True) * lane_m

    @pl.when(t == 0)
    def _init():
        imp_ref[...] = imp_t
        load_ref[...] = load_t

    @pl.when(t > 0)
    def _acc():
        imp_ref[...] = imp_ref[...] + imp_t
        load_ref[...] = load_ref[...] + load_t

    @pl.when(t == NTL - 1)
    def _loss():
        def cv2(v):
            mean = jnp.sum(v) / E
            var = jnp.sum(jnp.where(lane_m > 0, (v - mean) ** 2,
                                    0.0)) / (E - 1)
            return var / (mean * mean + 1e-10)

        loss_ref[0, 0] = (cv2(imp_ref[...]) + cv2(load_ref[...])) * 0.01


@jax.jit
def _moe(data, w_gate_p, W1b, b1a, W2a, b2p):
    y, loss = pl.pallas_call(
        _moe_kernel,
        grid=(NTL,),
        in_specs=[
            pl.BlockSpec((TN, D), lambda t: (t, 0)),
            pl.BlockSpec((TN, D), lambda t: (t, 0)),
            pl.BlockSpec((D, EP), lambda t: (0, 0)),
            pl.BlockSpec((E, D, F), lambda t: (0, 0, 0)),
            pl.BlockSpec((1, 1, E * F), lambda t: (0, 0, 0)),
            pl.BlockSpec((E * F, D), lambda t: (0, 0)),
            pl.BlockSpec((EP, D), lambda t: (0, 0)),
        ],
        out_specs=[
            pl.BlockSpec((TN, D), lambda t: (t, 0)),
            pl.BlockSpec(memory_space=pltpu.SMEM),
        ],
        out_shape=[
            jax.ShapeDtypeStruct((N, D), jnp.float32),
            jax.ShapeDtypeStruct((1, 1), jnp.float32),
        ],
        scratch_shapes=[
            pltpu.VMEM((TN, E * F), jnp.bfloat16),
            pltpu.VMEM((1, EP), jnp.float32),
            pltpu.VMEM((1, EP), jnp.float32),
        ],
        compiler_params=pltpu.CompilerParams(
            dimension_semantics=("arbitrary",),
        ),
    )(data, data.astype(jnp.bfloat16), w_gate_p, W1b, b1a, W2a, b2p)
    return y, loss[0, 0]


def kernel(data, w_gate, w_noise, W1, b1, W2, b2):
    del w_noise
    w_gate_p = jnp.pad(w_gate, ((0, 0), (0, EP - E)))
    W1b = W1.astype(jnp.bfloat16)
    b1a = b1.reshape(1, 1, E * F)
    W2a = W2.astype(jnp.bfloat16).reshape(E * F, D)
    b2p = jnp.pad(b2, ((0, EP - E), (0, 0)))
    return _moe(data, w_gate_p, W1b, b1a, W2a, b2p)


# speedup vs baseline: 3.5829x; 1.2520x over previous
import jax
import jax.numpy as jnp
from jax.experimental import pallas as pl
from jax.experimental.pallas import tpu as pltpu

E = 8
K = 2
N = 2048
D = 768
F = 768
EP = 128


def _moe_fused_kernel(data_ref, wg_ref, w1_ref, b1_ref, w2_ref, b2_ref,
                      y_ref, loss_ref, gates_ref):
    e = pl.program_id(0)

    @pl.when(e == 0)
    def _gating():
        x = data_ref[...]
        logits = jnp.dot(x, wg_ref[...], preferred_element_type=jnp.float32)
        lane = jax.lax.broadcasted_iota(jnp.int32, (N, EP), 1)
        neg = jnp.float32(-jnp.inf)
        logits = jnp.where(lane < E, logits, neg)
        l1 = jnp.max(logits, axis=1, keepdims=True)
        a1 = jnp.min(jnp.where(logits == l1, lane, EP), axis=1, keepdims=True)
        m = jnp.where(lane == a1, neg, logits)
        l2 = jnp.max(m, axis=1, keepdims=True)
        a2 = jnp.min(jnp.where(m == l2, lane, EP), axis=1, keepdims=True)
        e2 = jnp.exp(l2 - l1)
        denom = 1.0 + e2
        g1 = 1.0 / denom
        g2 = e2 / denom
        gates = (jnp.where(lane == a1, g1, 0.0)
                 + jnp.where(lane == a2, g2, 0.0))
        gates_ref[...] = gates
        lane_m = (lane[0:1, :] < E).astype(jnp.float32)
        importance = jnp.sum(gates, axis=0, keepdims=True) * lane_m
        load = jnp.sum((gates > 0.0).astype(jnp.float32), axis=0,
                       keepdims=True) * lane_m

        def cv2(v):
            mean = jnp.sum(v) / E
            var = jnp.sum(jnp.where(lane_m > 0, (v - mean) ** 2, 0.0)) / (E - 1)
            return var / (mean * mean + 1e-10)

        loss_ref[0, 0] = (cv2(importance) + cv2(load)) * 0.01

    x = data_ref[...]
    h = jnp.dot(x, w1_ref[0], preferred_element_type=jnp.float32)
    h = jnp.maximum(h + b1_ref[0], 0.0)
    o = jnp.dot(h, w2_ref[0], preferred_element_type=jnp.float32)
    o = o + b2_ref[0]
    lane = jax.lax.broadcasted_iota(jnp.int32, (N, EP), 1)
    gcol = jnp.sum(jnp.where(lane == e, gates_ref[...], 0.0), axis=1,
                   keepdims=True)
    contrib = o * gcol

    @pl.when(e == 0)
    def _init():
        y_ref[...] = contrib

    @pl.when(e > 0)
    def _acc():
        y_ref[...] = y_ref[...] + contrib


@jax.jit
def _moe_fused(data, w_gate_p, W1, b1, W2, b2):
    y, loss = pl.pallas_call(
        _moe_fused_kernel,
        grid=(E,),
        in_specs=[
            pl.BlockSpec((N, D), lambda e: (0, 0)),
            pl.BlockSpec((D, EP), lambda e: (0, 0)),
            pl.BlockSpec((1, D, F), lambda e: (e, 0, 0)),
            pl.BlockSpec((1, 1, F), lambda e: (e, 0, 0)),
            pl.BlockSpec((1, F, D), lambda e: (e, 0, 0)),
            pl.BlockSpec((1, 1, D), lambda e: (e, 0, 0)),
        ],
        out_specs=[
            pl.BlockSpec((N, D), lambda e: (0, 0)),
            pl.BlockSpec(memory_space=pltpu.SMEM),
        ],
        out_shape=[
            jax.ShapeDtypeStruct((N, D), jnp.float32),
            jax.ShapeDtypeStruct((1, 1), jnp.float32),
        ],
        scratch_shapes=[pltpu.VMEM((N, EP), jnp.float32)],
        compiler_params=pltpu.CompilerParams(
            dimension_semantics=("arbitrary",),
        ),
    )(data, w_gate_p, W1, b1, W2, b2)
    return y, loss[0, 0]


def kernel(data, w_gate, w_noise, W1, b1, W2, b2):
    del w_noise
    w_gate_p = jnp.pad(w_gate, ((0, 0), (0, EP - E)))
    return _moe_fused(data, w_gate_p, W1, b1[:, None, :], W2, b2[:, None, :])
